# baseline (speedup 1.0000x reference)
import jax
import jax.numpy as jnp
from jax.experimental import pallas as pl

_HIDDEN = 2048
_EXPERTS = 64
_TOPK = 8
_BLOCK = 2048
_CHUNK = 128


def _router_kernel(x_ref, wt_ref, b_ref, scores_ref, idx_ref):
    x = x_ref[...]
    logits = jax.lax.dot_general(
        x, wt_ref[...], (((1,), (1,)), ((), ())),
        preferred_element_type=jnp.float32,
    )
    logits = logits + b_ref[...]

    for c in range(_BLOCK // _CHUNK):
        lt = logits[c * _CHUNK:(c + 1) * _CHUNK, :].T
        iota_s = jax.lax.broadcasted_iota(
            jnp.int32, lt.shape, 0).astype(jnp.float32)
        work = lt
        rowmax = None
        denom = None
        idx_rows = []
        for _ in range(_TOPK):
            vmax = jnp.max(work, axis=0, keepdims=True)
            if rowmax is None:
                rowmax = vmax
            is_max = work == vmax
            idx_k = jnp.min(jnp.where(is_max, iota_s, float(_EXPERTS)),
                            axis=0, keepdims=True)
            sel = iota_s == idx_k
            work = jnp.where(sel, -jnp.inf, work)
            idx_rows.append(idx_k)
            w = jnp.exp(vmax - rowmax)
            denom = w if denom is None else denom + w
        recip = 1.0 / denom
        mask = work == -jnp.inf
        scores_t = jnp.where(mask, jnp.exp(lt - rowmax) * recip, 0.0)
        scores_ref[c * _CHUNK:(c + 1) * _CHUNK, :] = scores_t.T
        idx_t = jnp.concatenate(idx_rows, axis=0)
        idx_ref[c * _CHUNK:(c + 1) * _CHUNK, :] = idx_t.T.astype(jnp.int32)


def kernel(hidden_states, W, b):
    x = hidden_states.reshape(-1, _HIDDEN)
    n = x.shape[0]
    b2 = b.reshape(1, _EXPERTS)
    scores, idx = pl.pallas_call(
        _router_kernel,
        grid=(n // _BLOCK,),
        in_specs=[
            pl.BlockSpec((_BLOCK, _HIDDEN), lambda i: (i, 0)),
            pl.BlockSpec((_EXPERTS, _HIDDEN), lambda i: (0, 0)),
            pl.BlockSpec((1, _EXPERTS), lambda i: (0, 0)),
        ],
        out_specs=[
            pl.BlockSpec((_BLOCK, _EXPERTS), lambda i: (i, 0)),
            pl.BlockSpec((_BLOCK, _TOPK), lambda i: (i, 0)),
        ],
        out_shape=[
            jax.ShapeDtypeStruct((n, _EXPERTS), jnp.float32),
            jax.ShapeDtypeStruct((n, _TOPK), jnp.int32),
        ],
    )(x, W, b2)
    return (scores, idx)
